# initial kernel scaffold (unmeasured)
import jax
import jax.numpy as jnp
from jax import lax
from jax.experimental import pallas as pl
from jax.experimental.pallas import tpu as pltpu

N_DEV = 8
T = 512
V_PER = 4096
D = 512

_MASKS = (1, 3, 4)


def kernel(ids, E):
    ids2d = ids.reshape(T, 1)

    def body(ids_ref, e_ref, out_ref, accum, recv_bufs, send_sems, recv_sems):
        my_pos = lax.axis_index("i")

        base = my_pos * V_PER
        local_ids = ids_ref[:, :] - base
        cols = lax.broadcasted_iota(jnp.int32, (T, V_PER), 1)
        onehot = (local_ids == cols).astype(jnp.bfloat16)
        partial = jax.lax.dot_general(
            onehot,
            e_ref[:, :].astype(jnp.bfloat16),
            (((1,), (0,)), ((), ())),
            preferred_element_type=jnp.float32,
        )
        accum[:, :] = partial.astype(jnp.bfloat16)

        for h, m in enumerate(_MASKS):
            partner = my_pos ^ m
            rdma = pltpu.make_async_remote_copy(
                src_ref=accum,
                dst_ref=recv_bufs.at[h],
                send_sem=send_sems.at[h],
                recv_sem=recv_sems.at[h],
                device_id=(partner,),
                device_id_type=pl.DeviceIdType.MESH,
            )
            rdma.start()
            rdma.wait()
            accum[:, :] = accum[:, :] + recv_bufs[h, :, :]

        out_ref[:, :] = accum[:, :].astype(jnp.float32)

    return pl.pallas_call(
        body,
        out_shape=jax.ShapeDtypeStruct((T, D), jnp.float32),
        in_specs=[
            pl.BlockSpec(memory_space=pltpu.VMEM),
            pl.BlockSpec(memory_space=pltpu.VMEM),
        ],
        out_specs=pl.BlockSpec(memory_space=pltpu.VMEM),
        scratch_shapes=[
            pltpu.VMEM((T, D), jnp.bfloat16),
            pltpu.VMEM((3, T, D), jnp.bfloat16),
            pltpu.SemaphoreType.DMA((3,)),
            pltpu.SemaphoreType.DMA((3,)),
        ],
        compiler_params=pltpu.CompilerParams(collective_id=0),
    )(ids2d, E)


# baseline (device time: 37854 ns/iter reference)
import jax
import jax.numpy as jnp
from jax import lax
from jax.experimental import pallas as pl
from jax.experimental.pallas import tpu as pltpu

N_DEV = 8
T = 512
V_PER = 4096
D = 512

_MASKS = (1, 3, 4)


def kernel(ids, E):
    ids2d = ids.reshape(T, 1)

    def body(ids_ref, e_ref, out_ref, accum, recv_bufs, send_sems, recv_sems):
        my_pos = lax.axis_index("i")

        base = my_pos * V_PER
        local_ids = ids_ref[:, :] - base
        cols = lax.broadcasted_iota(jnp.int32, (T, V_PER), 1)
        onehot = (local_ids == cols).astype(jnp.bfloat16)
        partial = jax.lax.dot_general(
            onehot,
            e_ref[:, :].astype(jnp.bfloat16),
            (((1,), (0,)), ((), ())),
            preferred_element_type=jnp.float32,
        )
        accum[:, :] = partial.astype(jnp.bfloat16)

        for h, m in enumerate(_MASKS):
            partner = my_pos ^ m
            rdma = pltpu.make_async_remote_copy(
                src_ref=accum,
                dst_ref=recv_bufs.at[h],
                send_sem=send_sems.at[h],
                recv_sem=recv_sems.at[h],
                device_id=(partner,),
                device_id_type=pl.DeviceIdType.MESH,
            )
            rdma.start()
            rdma.wait()
            accum[:, :] = accum[:, :] + recv_bufs[h, :, :]

        out_ref[:, :] = accum[:, :].astype(jnp.float32)

    return pl.pallas_call(
        body,
        out_shape=jax.ShapeDtypeStruct((T, D), jnp.float32),
        in_specs=[
            pl.BlockSpec(memory_space=pltpu.VMEM),
            pl.BlockSpec(memory_space=pltpu.VMEM),
        ],
        out_specs=pl.BlockSpec(memory_space=pltpu.VMEM),
        scratch_shapes=[
            pltpu.VMEM((T, D), jnp.bfloat16),
            pltpu.VMEM((3, T, D), jnp.bfloat16),
            pltpu.SemaphoreType.DMA((3,)),
            pltpu.SemaphoreType.DMA((3,)),
        ],
    )(ids2d, E)


# device time: 22546 ns/iter; 1.6790x vs baseline; 1.6790x over previous
import jax
import jax.numpy as jnp
from jax import lax
from jax.experimental import pallas as pl
from jax.experimental.pallas import tpu as pltpu

N_DEV = 8
T = 512
V_PER = 4096
D = 512

_MASKS = (1, 3, 4)

_CHUNKS = ((0, 176), (176, 176), (352, 160))


def kernel(ids, E):
    ids2d = ids.reshape(T, 1)

    def body(ids_ref, e_ref, out_ref, accum, recv_bufs, send_sems, recv_sems):
        my_pos = lax.axis_index("i")

        barrier_sem = pltpu.get_barrier_semaphore()
        for m in _MASKS:
            pl.semaphore_signal(
                barrier_sem, inc=1,
                device_id=(my_pos ^ m,), device_id_type=pl.DeviceIdType.MESH,
            )
        pl.semaphore_wait(barrier_sem, 3)

        base = my_pos * V_PER
        local_ids = ids_ref[:, :] - base
        cols = lax.broadcasted_iota(jnp.int32, (T, V_PER), 1)
        onehot = (local_ids == cols).astype(jnp.bfloat16)
        partial = jax.lax.dot_general(
            onehot,
            e_ref[:, :].astype(jnp.bfloat16),
            (((1,), (0,)), ((), ())),
            preferred_element_type=jnp.float32,
        )
        accum[:, :] = partial.astype(jnp.bfloat16)

        for h in range(3):
            rdmas = []
            for j, (s, n) in enumerate(_CHUNKS):
                partner = my_pos ^ _MASKS[(h + j) % 3]
                rdma = pltpu.make_async_remote_copy(
                    src_ref=accum.at[pl.ds(s, n), :],
                    dst_ref=recv_bufs.at[h, pl.ds(s, n), :],
                    send_sem=send_sems.at[h, j],
                    recv_sem=recv_sems.at[h, j],
                    device_id=(partner,),
                    device_id_type=pl.DeviceIdType.MESH,
                )
                rdma.start()
                rdmas.append(rdma)
            for rdma in rdmas:
                rdma.wait()
            accum[:, :] = accum[:, :] + recv_bufs[h, :, :]

        out_ref[:, :] = accum[:, :].astype(jnp.float32)

    return pl.pallas_call(
        body,
        out_shape=jax.ShapeDtypeStruct((T, D), jnp.float32),
        in_specs=[
            pl.BlockSpec(memory_space=pltpu.VMEM),
            pl.BlockSpec(memory_space=pltpu.VMEM),
        ],
        out_specs=pl.BlockSpec(memory_space=pltpu.VMEM),
        scratch_shapes=[
            pltpu.VMEM((T, D), jnp.bfloat16),
            pltpu.VMEM((3, T, D), jnp.bfloat16),
            pltpu.SemaphoreType.DMA((3, 3)),
            pltpu.SemaphoreType.DMA((3, 3)),
        ],
        compiler_params=pltpu.CompilerParams(collective_id=0),
    )(ids2d, E)


# device time: 22276 ns/iter; 1.6993x vs baseline; 1.0121x over previous
import jax
import jax.numpy as jnp
from jax import lax
from jax.experimental import pallas as pl
from jax.experimental.pallas import tpu as pltpu

N_DEV = 8
T = 512
V_PER = 4096
D = 512

_MASKS = (1, 3, 4)

_CHUNKS = ((0, 176), (176, 176), (352, 160))


def kernel(ids, E):
    ids2d = ids.reshape(T, 1)

    def body(ids_ref, e_ref, out_ref, accum, recv_bufs, send_sems, recv_sems):
        my_pos = lax.axis_index("i")

        barrier_sem = pltpu.get_barrier_semaphore()
        for m in _MASKS:
            pl.semaphore_signal(
                barrier_sem, inc=1,
                device_id=(my_pos ^ m,), device_id_type=pl.DeviceIdType.MESH,
            )
        pl.semaphore_wait(barrier_sem, 3)

        e_bf16 = e_ref[:, :].astype(jnp.bfloat16)
        base = my_pos * V_PER

        def exchange(h, j, s, n):
            partner = my_pos ^ _MASKS[(h + j) % 3]
            return pltpu.make_async_remote_copy(
                src_ref=accum.at[pl.ds(s, n), :],
                dst_ref=recv_bufs.at[h, pl.ds(s, n), :],
                send_sem=send_sems.at[h, j],
                recv_sem=recv_sems.at[h, j],
                device_id=(partner,),
                device_id_type=pl.DeviceIdType.MESH,
            )

        rdmas = {}
        for j, (s, n) in enumerate(_CHUNKS):
            local_ids = ids_ref[pl.ds(s, n), :] - base
            cols = lax.broadcasted_iota(jnp.int32, (n, V_PER), 1)
            onehot = (local_ids == cols).astype(jnp.bfloat16)
            partial = jax.lax.dot_general(
                onehot, e_bf16,
                (((1,), (0,)), ((), ())),
                preferred_element_type=jnp.float32,
            )
            accum[pl.ds(s, n), :] = partial.astype(jnp.bfloat16)
            rdmas[0, j] = exchange(0, j, s, n)
            rdmas[0, j].start()

        for h in range(2):
            for j, (s, n) in enumerate(_CHUNKS):
                rdmas[h, j].wait()
                accum[pl.ds(s, n), :] = (
                    accum[pl.ds(s, n), :] + recv_bufs[h, pl.ds(s, n), :]
                )
                rdmas[h + 1, j] = exchange(h + 1, j, s, n)
                rdmas[h + 1, j].start()

        for j, (s, n) in enumerate(_CHUNKS):
            rdmas[2, j].wait()
            out_ref[pl.ds(s, n), :] = (
                accum[pl.ds(s, n), :] + recv_bufs[2, pl.ds(s, n), :]
            ).astype(jnp.float32)

    return pl.pallas_call(
        body,
        out_shape=jax.ShapeDtypeStruct((T, D), jnp.float32),
        in_specs=[
            pl.BlockSpec(memory_space=pltpu.VMEM),
            pl.BlockSpec(memory_space=pltpu.VMEM),
        ],
        out_specs=pl.BlockSpec(memory_space=pltpu.VMEM),
        scratch_shapes=[
            pltpu.VMEM((T, D), jnp.bfloat16),
            pltpu.VMEM((3, T, D), jnp.bfloat16),
            pltpu.SemaphoreType.DMA((3, 3)),
            pltpu.SemaphoreType.DMA((3, 3)),
        ],
        compiler_params=pltpu.CompilerParams(collective_id=0),
    )(ids2d, E)
